# Initial kernel scaffold; baseline (speedup 1.0000x reference)
import jax
import jax.numpy as jnp
from jax.experimental import pallas as pl


def kernel(x, edge_index, W1, b1, W2, b2):
    raise NotImplementedError("write your pallas kernel here")


# speedup vs baseline: 21.6051x; 21.6051x over previous
import functools

import jax
import jax.numpy as jnp
from jax import lax
from jax.experimental import pallas as pl
from jax.experimental.pallas import tpu as pltpu
from jax.experimental.pallas import tpu_sc as plsc

_NC = 2
_NS = 16
_NW = _NC * _NS
_CHUNK = 128
_DEGW = 16


def _round_up(v, m):
    return (v + m - 1) // m * m


def _make_edge_scatter(n_pad, ch, d):
    rpt = n_pad // _NS
    mesh = plsc.VectorSubcoreMesh(core_axis_name="c", subcore_axis_name="s")

    @functools.partial(
        pl.kernel,
        out_type=jax.ShapeDtypeStruct((_NC, n_pad, d), jnp.float32),
        mesh=mesh,
        compiler_params=pltpu.CompilerParams(use_tc_tiling_on_sc=False),
        scratch_types=[
            pltpu.VMEM((ch, _CHUNK), jnp.int32),
            pltpu.VMEM((ch, _CHUNK), jnp.int32),
            pltpu.VMEM((_CHUNK, d), jnp.float32),
            pltpu.VMEM_SHARED((n_pad, d), jnp.float32),
            pltpu.SemaphoreType.DMA,
        ],
    )
    def k(table, srcg, dstg, zeros, out, src_idx, dst_idx, rows, acc, sem):
        cid = lax.axis_index("c")
        sid = lax.axis_index("s")
        wid = cid * _NS + sid
        pltpu.sync_copy(srcg.at[wid], src_idx)
        pltpu.sync_copy(dstg.at[wid], dst_idx)
        pltpu.sync_copy(zeros.at[pl.ds(sid * rpt, rpt)],
                        acc.at[pl.ds(sid * rpt, rpt)])
        plsc.subcore_barrier()

        def step(c, carry):
            pltpu.async_copy(table.at[src_idx.at[c]], rows, sem).wait()
            pltpu.sync_copy(rows, acc.at[dst_idx.at[c]], add=True)
            return carry

        lax.fori_loop(0, ch, step, 0)
        plsc.subcore_barrier()
        pltpu.sync_copy(acc.at[pl.ds(sid * rpt, rpt)],
                        out.at[cid, pl.ds(sid * rpt, rpt)])

    return k


def _make_deg_scatter(n_pad, ch):
    rpt = n_pad // _NS
    mesh = plsc.VectorSubcoreMesh(core_axis_name="c", subcore_axis_name="s")

    @functools.partial(
        pl.kernel,
        out_type=jax.ShapeDtypeStruct((_NC, n_pad, _DEGW), jnp.float32),
        mesh=mesh,
        compiler_params=pltpu.CompilerParams(use_tc_tiling_on_sc=False),
        scratch_types=[
            pltpu.VMEM((ch, _CHUNK), jnp.int32),
            pltpu.VMEM((_CHUNK, _DEGW), jnp.float32),
            pltpu.VMEM_SHARED((n_pad, _DEGW), jnp.float32),
        ],
    )
    def k(dstg, ones, zeros, out, dst_idx, ones_buf, acc):
        cid = lax.axis_index("c")
        sid = lax.axis_index("s")
        wid = cid * _NS + sid
        pltpu.sync_copy(dstg.at[wid], dst_idx)
        pltpu.sync_copy(ones, ones_buf)
        pltpu.sync_copy(zeros.at[pl.ds(sid * rpt, rpt)],
                        acc.at[pl.ds(sid * rpt, rpt)])
        plsc.subcore_barrier()

        def step(c, carry):
            pltpu.sync_copy(ones_buf, acc.at[dst_idx.at[c]], add=True)
            return carry

        lax.fori_loop(0, ch, step, 0)
        plsc.subcore_barrier()
        pltpu.sync_copy(acc.at[pl.ds(sid * rpt, rpt)],
                        out.at[cid, pl.ds(sid * rpt, rpt)])

    return k


def _tc_prep(degp_ref, x_ref, w1_ref, g1_ref, dinv_ref):
    deg = degp_ref[0, :, :1] + degp_ref[1, :, :1] + 1.0
    dinv = lax.rsqrt(deg)
    dinv_ref[...] = dinv
    g1_ref[...] = jnp.dot(x_ref[...] * dinv, w1_ref[...],
                          preferred_element_type=jnp.float32)


def _tc_mid(p_ref, g1_ref, dinv_ref, b1_ref, w2_ref, g2_ref):
    s = p_ref[0] + p_ref[1] + g1_ref[...]
    h = jnp.maximum(s * dinv_ref[...] + b1_ref[...], 0.0)
    g2_ref[...] = jnp.dot(h, w2_ref[...],
                          preferred_element_type=jnp.float32) * dinv_ref[...]


def _tc_final(p_ref, g2_ref, dinv_ref, b2_ref, o_ref):
    o_ref[...] = ((p_ref[0] + p_ref[1] + g2_ref[...]) * dinv_ref[...]
                  + b2_ref[...])


def kernel(x, edge_index, W1, b1, W2, b2):
    n, d_in = x.shape
    e = edge_index.shape[1]
    d_h = W1.shape[1]
    d_out = W2.shape[1]
    n_pad = _round_up(n, 256)
    e_pad = _round_up(e, _NW * _CHUNK)
    ch = e_pad // (_NW * _CHUNK)

    src = edge_index[0]
    dst = edge_index[1]
    pad = e_pad - e
    srcp = jnp.concatenate(
        [src, jnp.zeros((pad,), jnp.int32)]).reshape(_NW, ch, _CHUNK)
    dstp = jnp.concatenate(
        [dst, jnp.full((pad,), n, jnp.int32)]).reshape(_NW, ch, _CHUNK)
    xp = jnp.pad(x, ((0, n_pad - n), (0, 0)))

    ones = jnp.ones((_CHUNK, _DEGW), jnp.float32)
    z_deg = jnp.zeros((n_pad, _DEGW), jnp.float32)
    z_h = jnp.zeros((n_pad, d_h), jnp.float32)
    z_o = jnp.zeros((n_pad, d_out), jnp.float32)

    degp = _make_deg_scatter(n_pad, ch)(dstp, ones, z_deg)

    g1, dinv = pl.pallas_call(
        _tc_prep,
        out_shape=(jax.ShapeDtypeStruct((n_pad, d_h), jnp.float32),
                   jax.ShapeDtypeStruct((n_pad, 1), jnp.float32)),
    )(degp, xp, W1)

    p1 = _make_edge_scatter(n_pad, ch, d_h)(g1, srcp, dstp, z_h)

    g2 = pl.pallas_call(
        _tc_mid,
        out_shape=jax.ShapeDtypeStruct((n_pad, d_out), jnp.float32),
    )(p1, g1, dinv, b1.reshape(1, d_h), W2)

    p2 = _make_edge_scatter(n_pad, ch, d_out)(g2, srcp, dstp, z_o)

    outp = pl.pallas_call(
        _tc_final,
        out_shape=jax.ShapeDtypeStruct((n_pad, d_out), jnp.float32),
    )(p2, g2, dinv, b2.reshape(1, d_out))

    return outp[:n]


# speedup vs baseline: 29.2289x; 1.3529x over previous
import functools

import jax
import jax.numpy as jnp
from jax import lax
from jax.experimental import pallas as pl
from jax.experimental.pallas import tpu as pltpu
from jax.experimental.pallas import tpu_sc as plsc

_NC = 2
_NS = 16
_NW = _NC * _NS
_CHUNK = 128
_DEGW = 16


def _round_up(v, m):
    return (v + m - 1) // m * m


def _make_edge_scatter(n_pad, ch, d):
    rpt = n_pad // _NS
    mesh = plsc.VectorSubcoreMesh(core_axis_name="c", subcore_axis_name="s")

    @functools.partial(
        pl.kernel,
        out_type=jax.ShapeDtypeStruct((_NC, n_pad, d), jnp.float32),
        mesh=mesh,
        compiler_params=pltpu.CompilerParams(use_tc_tiling_on_sc=False),
        scratch_types=[
            pltpu.VMEM((ch, _CHUNK), jnp.int32),
            pltpu.VMEM((ch, _CHUNK), jnp.int32),
            pltpu.VMEM((4, _CHUNK, d), jnp.float32),
            pltpu.VMEM_SHARED((n_pad, d), jnp.float32),
            pltpu.SemaphoreType.DMA((4,)),
            pltpu.SemaphoreType.DMA((4,)),
        ],
    )
    def k(table, srcg, dstg, zeros, out, src_idx, dst_idx, rows, acc,
          gsem, ssem):
        cid = lax.axis_index("c")
        sid = lax.axis_index("s")
        wid = cid * _NS + sid
        pltpu.sync_copy(srcg.at[wid], src_idx)
        pltpu.sync_copy(dstg.at[wid], dst_idx)
        pltpu.sync_copy(zeros.at[pl.ds(sid * rpt, rpt)],
                        acc.at[pl.ds(sid * rpt, rpt)])
        plsc.subcore_barrier()

        pltpu.async_copy(table.at[src_idx.at[0]], rows.at[0], gsem.at[0])
        pltpu.async_copy(table.at[src_idx.at[1]], rows.at[1], gsem.at[1])

        def step(c, carry):
            bg = lax.rem(c + 2, 4)

            @pl.when(c + 2 < ch)
            def _():
                @pl.when(c >= 2)
                def _():
                    pltpu.make_async_copy(rows.at[bg],
                                          acc.at[dst_idx.at[c]],
                                          ssem.at[bg]).wait()
                pltpu.async_copy(table.at[src_idx.at[c + 2]], rows.at[bg],
                                 gsem.at[bg])

            b = lax.rem(c, 4)
            pltpu.make_async_copy(table.at[src_idx.at[c]], rows.at[b],
                                  gsem.at[b]).wait()
            pltpu.async_copy(rows.at[b], acc.at[dst_idx.at[c]], ssem.at[b],
                             add=True)
            return carry

        lax.fori_loop(0, ch, step, 0)

        def drain(j, carry):
            pltpu.make_async_copy(rows.at[lax.rem(j, 4)],
                                  acc.at[dst_idx.at[j]],
                                  ssem.at[lax.rem(j, 4)]).wait()
            return carry

        lax.fori_loop(ch - 4, ch, drain, 0)
        plsc.subcore_barrier()
        pltpu.sync_copy(acc.at[pl.ds(sid * rpt, rpt)],
                        out.at[cid, pl.ds(sid * rpt, rpt)])

    return k


def _make_deg_scatter(n_pad, ch):
    rpt = n_pad // _NS
    mesh = plsc.VectorSubcoreMesh(core_axis_name="c", subcore_axis_name="s")

    @functools.partial(
        pl.kernel,
        out_type=jax.ShapeDtypeStruct((_NC, n_pad, _DEGW), jnp.float32),
        mesh=mesh,
        compiler_params=pltpu.CompilerParams(use_tc_tiling_on_sc=False),
        scratch_types=[
            pltpu.VMEM((ch, _CHUNK), jnp.int32),
            pltpu.VMEM((_CHUNK, _DEGW), jnp.float32),
            pltpu.VMEM_SHARED((n_pad, _DEGW), jnp.float32),
            pltpu.SemaphoreType.DMA,
        ],
    )
    def k(dstg, ones, zeros, out, dst_idx, ones_buf, acc, ssem):
        cid = lax.axis_index("c")
        sid = lax.axis_index("s")
        wid = cid * _NS + sid
        pltpu.sync_copy(dstg.at[wid], dst_idx)
        pltpu.sync_copy(ones, ones_buf)
        pltpu.sync_copy(zeros.at[pl.ds(sid * rpt, rpt)],
                        acc.at[pl.ds(sid * rpt, rpt)])
        plsc.subcore_barrier()

        def step(c, carry):
            pltpu.async_copy(ones_buf, acc.at[dst_idx.at[c]], ssem, add=True)
            return carry

        lax.fori_loop(0, ch, step, 0)

        def drain(c, carry):
            pltpu.make_async_copy(ones_buf, acc.at[dst_idx.at[c]],
                                  ssem).wait()
            return carry

        lax.fori_loop(0, ch, drain, 0)
        plsc.subcore_barrier()
        pltpu.sync_copy(acc.at[pl.ds(sid * rpt, rpt)],
                        out.at[cid, pl.ds(sid * rpt, rpt)])

    return k


def _tc_prep(degp_ref, x_ref, w1_ref, g1_ref, dinv_ref):
    n = x_ref.shape[0]
    deg = degp_ref[0, :n, :1] + degp_ref[1, :n, :1] + 1.0
    dinv = lax.rsqrt(deg)
    dinv_ref[...] = dinv
    g1_ref[...] = jnp.dot(x_ref[...] * dinv, w1_ref[...],
                          preferred_element_type=jnp.float32)


def _tc_mid(p_ref, g1_ref, dinv_ref, b1_ref, w2_ref, g2_ref):
    n = g1_ref.shape[0]
    s = p_ref[0, :n] + p_ref[1, :n] + g1_ref[...]
    h = jnp.maximum(s * dinv_ref[...] + b1_ref[...], 0.0)
    g2_ref[...] = jnp.dot(h, w2_ref[...],
                          preferred_element_type=jnp.float32) * dinv_ref[...]


def _tc_final(p_ref, g2_ref, dinv_ref, b2_ref, o_ref):
    n = g2_ref.shape[0]
    o_ref[...] = ((p_ref[0, :n] + p_ref[1, :n] + g2_ref[...])
                  * dinv_ref[...] + b2_ref[...])


def kernel(x, edge_index, W1, b1, W2, b2):
    n, d_in = x.shape
    e = edge_index.shape[1]
    d_h = W1.shape[1]
    d_out = W2.shape[1]
    n_pad = _round_up(n, 256)
    e_pad = _round_up(e, _NW * _CHUNK)
    ch = e_pad // (_NW * _CHUNK)

    src = edge_index[0]
    dst = edge_index[1]
    pad = e_pad - e
    srcp = jnp.concatenate(
        [src, jnp.zeros((pad,), jnp.int32)]).reshape(_NW, ch, _CHUNK)
    dstp = jnp.concatenate(
        [dst, jnp.full((pad,), n, jnp.int32)]).reshape(_NW, ch, _CHUNK)

    ones = jnp.ones((_CHUNK, _DEGW), jnp.float32)
    z_deg = jnp.zeros((n_pad, _DEGW), jnp.float32)
    z_h = jnp.zeros((n_pad, d_h), jnp.float32)
    z_o = jnp.zeros((n_pad, d_out), jnp.float32)

    degp = _make_deg_scatter(n_pad, ch)(dstp, ones, z_deg)

    g1, dinv = pl.pallas_call(
        _tc_prep,
        out_shape=(jax.ShapeDtypeStruct((n, d_h), jnp.float32),
                   jax.ShapeDtypeStruct((n, 1), jnp.float32)),
    )(degp, x, W1)

    p1 = _make_edge_scatter(n_pad, ch, d_h)(g1, srcp, dstp, z_h)

    g2 = pl.pallas_call(
        _tc_mid,
        out_shape=jax.ShapeDtypeStruct((n, d_out), jnp.float32),
    )(p1, g1, dinv, b1.reshape(1, d_h), W2)

    p2 = _make_edge_scatter(n_pad, ch, d_out)(g2, srcp, dstp, z_o)

    return pl.pallas_call(
        _tc_final,
        out_shape=jax.ShapeDtypeStruct((n, d_out), jnp.float32),
    )(p2, g2, dinv, b2.reshape(1, d_out))


# speedup vs baseline: 32.2530x; 1.1035x over previous
import functools

import jax
import jax.numpy as jnp
from jax import lax
from jax.experimental import pallas as pl
from jax.experimental.pallas import tpu as pltpu
from jax.experimental.pallas import tpu_sc as plsc

_NC = 2
_NS = 16
_NW = _NC * _NS
_CHUNK = 128
_DEGW = 16


def _round_up(v, m):
    return (v + m - 1) // m * m


def _make_edge_scatter(n, n_pad, ch, d, stage_table):
    rpt = n_pad // _NS
    tpt = n // _NS
    mesh = plsc.VectorSubcoreMesh(core_axis_name="c", subcore_axis_name="s")

    @functools.partial(
        pl.kernel,
        out_type=jax.ShapeDtypeStruct((_NC, n_pad, d), jnp.float32),
        mesh=mesh,
        compiler_params=pltpu.CompilerParams(use_tc_tiling_on_sc=False),
        scratch_types=[
            pltpu.VMEM((ch, _CHUNK), jnp.int32),
            pltpu.VMEM((ch, _CHUNK), jnp.int32),
            pltpu.VMEM((4, _CHUNK, d), jnp.float32),
            pltpu.VMEM_SHARED((n_pad, d), jnp.float32),
        ] + ([pltpu.VMEM_SHARED((n, d), jnp.float32)] if stage_table else [])
        + [
            pltpu.SemaphoreType.DMA((4,)),
            pltpu.SemaphoreType.DMA((4,)),
        ],
    )
    def k(table, srcg, dstg, out, src_idx, dst_idx, rows, acc, *rest):
        if stage_table:
            tbl, gsem, ssem = rest
        else:
            (gsem, ssem), tbl = rest, table
        cid = lax.axis_index("c")
        sid = lax.axis_index("s")
        wid = cid * _NS + sid
        pltpu.sync_copy(srcg.at[wid], src_idx)
        pltpu.sync_copy(dstg.at[wid], dst_idx)
        if stage_table:
            pltpu.sync_copy(table.at[pl.ds(sid * tpt, tpt)],
                            tbl.at[pl.ds(sid * tpt, tpt)])
        zv = jnp.zeros((16,), jnp.float32)

        def zstore(i, carry):
            rows[0, i // (d // 16), pl.ds((i % (d // 16)) * 16, 16)] = zv
            return carry

        lax.fori_loop(0, _CHUNK * d // 16, zstore, 0)

        def zcopy(i, carry):
            pltpu.sync_copy(rows.at[0],
                            acc.at[pl.ds(sid * rpt + i * _CHUNK, _CHUNK)])
            return carry

        lax.fori_loop(0, rpt // _CHUNK, zcopy, 0)
        plsc.subcore_barrier()

        pltpu.async_copy(tbl.at[src_idx.at[0]], rows.at[0], gsem.at[0])
        pltpu.async_copy(tbl.at[src_idx.at[1]], rows.at[1], gsem.at[1])

        def step(c, carry):
            bg = lax.rem(c + 2, 4)

            @pl.when(c + 2 < ch)
            def _():
                @pl.when(c >= 2)
                def _():
                    pltpu.make_async_copy(rows.at[bg],
                                          acc.at[dst_idx.at[c]],
                                          ssem.at[bg]).wait()
                pltpu.async_copy(tbl.at[src_idx.at[c + 2]], rows.at[bg],
                                 gsem.at[bg])

            b = lax.rem(c, 4)
            pltpu.make_async_copy(tbl.at[src_idx.at[c]], rows.at[b],
                                  gsem.at[b]).wait()
            pltpu.async_copy(rows.at[b], acc.at[dst_idx.at[c]], ssem.at[b],
                             add=True)
            return carry

        lax.fori_loop(0, ch, step, 0)

        def drain(j, carry):
            pltpu.make_async_copy(rows.at[lax.rem(j, 4)],
                                  acc.at[dst_idx.at[j]],
                                  ssem.at[lax.rem(j, 4)]).wait()
            return carry

        lax.fori_loop(ch - 4, ch, drain, 0)
        plsc.subcore_barrier()
        pltpu.sync_copy(acc.at[pl.ds(sid * rpt, rpt)],
                        out.at[cid, pl.ds(sid * rpt, rpt)])

    return k


def _make_deg_scatter(n_pad, ch):
    rpt = n_pad // _NS
    mesh = plsc.VectorSubcoreMesh(core_axis_name="c", subcore_axis_name="s")

    @functools.partial(
        pl.kernel,
        out_type=jax.ShapeDtypeStruct((_NC, n_pad, _DEGW), jnp.float32),
        mesh=mesh,
        compiler_params=pltpu.CompilerParams(use_tc_tiling_on_sc=False),
        scratch_types=[
            pltpu.VMEM((ch, _CHUNK), jnp.int32),
            pltpu.VMEM((_CHUNK, _DEGW), jnp.float32),
            pltpu.VMEM_SHARED((n_pad, _DEGW), jnp.float32),
            pltpu.SemaphoreType.DMA,
        ],
    )
    def k(dstg, ones, zeros, out, dst_idx, ones_buf, acc, ssem):
        cid = lax.axis_index("c")
        sid = lax.axis_index("s")
        wid = cid * _NS + sid
        pltpu.sync_copy(dstg.at[wid], dst_idx)
        pltpu.sync_copy(ones, ones_buf)
        pltpu.sync_copy(zeros.at[pl.ds(sid * rpt, rpt)],
                        acc.at[pl.ds(sid * rpt, rpt)])
        plsc.subcore_barrier()

        def step(c, carry):
            pltpu.async_copy(ones_buf, acc.at[dst_idx.at[c]], ssem, add=True)
            return carry

        lax.fori_loop(0, ch, step, 0)

        def drain(c, carry):
            pltpu.make_async_copy(ones_buf, acc.at[dst_idx.at[c]],
                                  ssem).wait()
            return carry

        lax.fori_loop(0, ch, drain, 0)
        plsc.subcore_barrier()
        pltpu.sync_copy(acc.at[pl.ds(sid * rpt, rpt)],
                        out.at[cid, pl.ds(sid * rpt, rpt)])

    return k


def _tc_prep(degp_ref, x_ref, w1_ref, g1_ref, dinv_ref):
    n = x_ref.shape[0]
    deg = degp_ref[0, :n, :1] + degp_ref[1, :n, :1] + 1.0
    dinv = lax.rsqrt(deg)
    dinv_ref[...] = dinv
    g1_ref[...] = jnp.dot(x_ref[...] * dinv, w1_ref[...],
                          preferred_element_type=jnp.float32)


def _tc_mid(p_ref, g1_ref, dinv_ref, b1_ref, w2_ref, g2_ref):
    n = g1_ref.shape[0]
    s = p_ref[0, :n] + p_ref[1, :n] + g1_ref[...]
    h = jnp.maximum(s * dinv_ref[...] + b1_ref[...], 0.0)
    g2_ref[...] = jnp.dot(h, w2_ref[...],
                          preferred_element_type=jnp.float32) * dinv_ref[...]


def _tc_final(p_ref, g2_ref, dinv_ref, b2_ref, o_ref):
    n = g2_ref.shape[0]
    o_ref[...] = ((p_ref[0, :n] + p_ref[1, :n] + g2_ref[...])
                  * dinv_ref[...] + b2_ref[...])


def kernel(x, edge_index, W1, b1, W2, b2):
    n, d_in = x.shape
    e = edge_index.shape[1]
    d_h = W1.shape[1]
    d_out = W2.shape[1]
    n_pad = _round_up(n, 256)
    e_pad = _round_up(e, _NW * _CHUNK)
    ch = e_pad // (_NW * _CHUNK)

    src = edge_index[0]
    dst = edge_index[1]
    pad = e_pad - e
    srcp = jnp.concatenate(
        [src, jnp.zeros((pad,), jnp.int32)]).reshape(_NW, ch, _CHUNK)
    dstp = jnp.concatenate(
        [dst, jnp.full((pad,), n, jnp.int32)]).reshape(_NW, ch, _CHUNK)

    ones = jnp.ones((_CHUNK, _DEGW), jnp.float32)
    z_deg = jnp.zeros((n_pad, _DEGW), jnp.float32)

    degp = _make_deg_scatter(n_pad, ch)(dstp, ones, z_deg)

    g1, dinv = pl.pallas_call(
        _tc_prep,
        out_shape=(jax.ShapeDtypeStruct((n, d_h), jnp.float32),
                   jax.ShapeDtypeStruct((n, 1), jnp.float32)),
    )(degp, x, W1)

    p1 = _make_edge_scatter(n, n_pad, ch, d_h, False)(g1, srcp, dstp)

    g2 = pl.pallas_call(
        _tc_mid,
        out_shape=jax.ShapeDtypeStruct((n, d_out), jnp.float32),
    )(p1, g1, dinv, b1.reshape(1, d_h), W2)

    p2 = _make_edge_scatter(n, n_pad, ch, d_out, True)(g2, srcp, dstp)

    return pl.pallas_call(
        _tc_final,
        out_shape=jax.ShapeDtypeStruct((n, d_out), jnp.float32),
    )(p2, g2, dinv, b2.reshape(1, d_out))


# speedup vs baseline: 43.0436x; 1.3346x over previous
import functools

import jax
import jax.numpy as jnp
from jax import lax
from jax.experimental import pallas as pl
from jax.experimental.pallas import tpu as pltpu
from jax.experimental.pallas import tpu_sc as plsc

_NC = 2
_NS = 16
_NW = _NC * _NS
_CHUNK = 128
_DEGW = 16


def _round_up(v, m):
    return (v + m - 1) // m * m


def _make_edge_scatter(n, n_pad, ch, d, col_split):
    rpt = n_pad // _NS
    tpt = n // _NS
    mesh = plsc.VectorSubcoreMesh(core_axis_name="c", subcore_axis_name="s")

    @functools.partial(
        pl.kernel,
        out_type=jax.ShapeDtypeStruct((_NC, n_pad, d), jnp.float32),
        mesh=mesh,
        compiler_params=pltpu.CompilerParams(use_tc_tiling_on_sc=False),
        scratch_types=[
            pltpu.VMEM((ch, _CHUNK), jnp.int32),
            pltpu.VMEM((ch, _CHUNK), jnp.int32),
            pltpu.VMEM((4, _CHUNK, d), jnp.float32),
            pltpu.VMEM_SHARED((n_pad, d), jnp.float32),
            pltpu.VMEM_SHARED((n, d), jnp.float32),
            pltpu.SemaphoreType.DMA((4,)),
            pltpu.SemaphoreType.DMA((4,)),
        ],
    )
    def k(table, srcg, dstg, out, src_idx, dst_idx, rows, acc, tbl,
          gsem, ssem):
        cid = lax.axis_index("c")
        sid = lax.axis_index("s")
        gid = sid if col_split else cid * _NS + sid
        pltpu.sync_copy(srcg.at[gid], src_idx)
        pltpu.sync_copy(dstg.at[gid], dst_idx)
        tsrc = table.at[cid] if col_split else table
        pltpu.sync_copy(tsrc.at[pl.ds(sid * tpt, tpt)],
                        tbl.at[pl.ds(sid * tpt, tpt)])
        zv = jnp.zeros((16,), jnp.float32)

        def zstore(i, carry):
            rows[0, i // (d // 16), pl.ds((i % (d // 16)) * 16, 16)] = zv
            return carry

        lax.fori_loop(0, _CHUNK * d // 16, zstore, 0)

        def zcopy(i, carry):
            pltpu.sync_copy(rows.at[0],
                            acc.at[pl.ds(sid * rpt + i * _CHUNK, _CHUNK)])
            return carry

        lax.fori_loop(0, rpt // _CHUNK, zcopy, 0)
        plsc.subcore_barrier()

        pltpu.async_copy(tbl.at[src_idx.at[0]], rows.at[0], gsem.at[0])
        pltpu.async_copy(tbl.at[src_idx.at[1]], rows.at[1], gsem.at[1])

        def step(c, carry):
            bg = lax.rem(c + 2, 4)

            @pl.when(c + 2 < ch)
            def _():
                @pl.when(c >= 2)
                def _():
                    pltpu.make_async_copy(rows.at[bg],
                                          acc.at[dst_idx.at[c]],
                                          ssem.at[bg]).wait()
                pltpu.async_copy(tbl.at[src_idx.at[c + 2]], rows.at[bg],
                                 gsem.at[bg])

            b = lax.rem(c, 4)
            pltpu.make_async_copy(tbl.at[src_idx.at[c]], rows.at[b],
                                  gsem.at[b]).wait()
            pltpu.async_copy(rows.at[b], acc.at[dst_idx.at[c]], ssem.at[b],
                             add=True)
            return carry

        lax.fori_loop(0, ch, step, 0)

        def drain(j, carry):
            pltpu.make_async_copy(rows.at[lax.rem(j, 4)],
                                  acc.at[dst_idx.at[j]],
                                  ssem.at[lax.rem(j, 4)]).wait()
            return carry

        lax.fori_loop(ch - 4, ch, drain, 0)
        plsc.subcore_barrier()
        pltpu.sync_copy(acc.at[pl.ds(sid * rpt, rpt)],
                        out.at[cid, pl.ds(sid * rpt, rpt)])

    return k


def _make_deg_scatter(n_pad, ch):
    rpt = n_pad // _NS
    mesh = plsc.VectorSubcoreMesh(core_axis_name="c", subcore_axis_name="s")

    @functools.partial(
        pl.kernel,
        out_type=jax.ShapeDtypeStruct((_NC, n_pad, _DEGW), jnp.float32),
        mesh=mesh,
        compiler_params=pltpu.CompilerParams(use_tc_tiling_on_sc=False),
        scratch_types=[
            pltpu.VMEM((ch, _CHUNK), jnp.int32),
            pltpu.VMEM((_CHUNK, _DEGW), jnp.float32),
            pltpu.VMEM_SHARED((n_pad, _DEGW), jnp.float32),
            pltpu.SemaphoreType.DMA,
        ],
    )
    def k(dstg, ones, zeros, out, dst_idx, ones_buf, acc, ssem):
        cid = lax.axis_index("c")
        sid = lax.axis_index("s")
        wid = cid * _NS + sid
        pltpu.sync_copy(dstg.at[wid], dst_idx)
        pltpu.sync_copy(ones, ones_buf)
        pltpu.sync_copy(zeros.at[pl.ds(sid * rpt, rpt)],
                        acc.at[pl.ds(sid * rpt, rpt)])
        plsc.subcore_barrier()

        def step(c, carry):
            pltpu.async_copy(ones_buf, acc.at[dst_idx.at[c]], ssem, add=True)
            return carry

        lax.fori_loop(0, ch, step, 0)

        def drain(c, carry):
            pltpu.make_async_copy(ones_buf, acc.at[dst_idx.at[c]],
                                  ssem).wait()
            return carry

        lax.fori_loop(0, ch, drain, 0)
        plsc.subcore_barrier()
        pltpu.sync_copy(acc.at[pl.ds(sid * rpt, rpt)],
                        out.at[cid, pl.ds(sid * rpt, rpt)])

    return k


def _tc_prep(degp_ref, x_ref, w1_ref, g1s_ref, dinv_ref):
    n = x_ref.shape[0]
    dh = g1s_ref.shape[2]
    deg = degp_ref[0, :n, :1] + degp_ref[1, :n, :1] + 1.0
    dinv = lax.rsqrt(deg)
    dinv_ref[...] = dinv
    g1 = jnp.dot(x_ref[...] * dinv, w1_ref[...],
                 preferred_element_type=jnp.float32)
    g1s_ref[0] = g1[:, :dh]
    g1s_ref[1] = g1[:, dh:]


def _tc_mid(p_ref, g1s_ref, dinv_ref, b1_ref, w2_ref, g2_ref):
    n = g1s_ref.shape[1]
    s = jnp.concatenate([p_ref[0, :n] + g1s_ref[0],
                         p_ref[1, :n] + g1s_ref[1]], axis=1)
    h = jnp.maximum(s * dinv_ref[...] + b1_ref[...], 0.0)
    g2_ref[...] = jnp.dot(h, w2_ref[...],
                          preferred_element_type=jnp.float32) * dinv_ref[...]


def _tc_final(p_ref, g2_ref, dinv_ref, b2_ref, o_ref):
    n = g2_ref.shape[0]
    o_ref[...] = ((p_ref[0, :n] + p_ref[1, :n] + g2_ref[...])
                  * dinv_ref[...] + b2_ref[...])


def kernel(x, edge_index, W1, b1, W2, b2):
    n, d_in = x.shape
    e = edge_index.shape[1]
    d_h = W1.shape[1]
    d_out = W2.shape[1]
    n_pad = _round_up(n, 256)
    e_pad = _round_up(e, _NW * _CHUNK)
    ch = e_pad // (_NW * _CHUNK)

    src = edge_index[0]
    dst = edge_index[1]
    pad = e_pad - e
    srcp = jnp.concatenate(
        [src, jnp.zeros((pad,), jnp.int32)]).reshape(_NW, ch, _CHUNK)
    dstp = jnp.concatenate(
        [dst, jnp.full((pad,), n, jnp.int32)]).reshape(_NW, ch, _CHUNK)

    ones = jnp.ones((_CHUNK, _DEGW), jnp.float32)
    z_deg = jnp.zeros((n_pad, _DEGW), jnp.float32)

    degp = _make_deg_scatter(n_pad, ch)(dstp, ones, z_deg)

    g1s, dinv = pl.pallas_call(
        _tc_prep,
        out_shape=(jax.ShapeDtypeStruct((2, n, d_h // 2), jnp.float32),
                   jax.ShapeDtypeStruct((n, 1), jnp.float32)),
    )(degp, x, W1)

    srcc = srcp.reshape(_NS, 2 * ch, _CHUNK)
    dstc = dstp.reshape(_NS, 2 * ch, _CHUNK)
    p1 = _make_edge_scatter(n, n_pad, 2 * ch, d_h // 2, True)(g1s, srcc, dstc)

    g2 = pl.pallas_call(
        _tc_mid,
        out_shape=jax.ShapeDtypeStruct((n, d_out), jnp.float32),
    )(p1, g1s, dinv, b1.reshape(1, d_h), W2)

    p2 = _make_edge_scatter(n, n_pad, ch, d_out, False)(g2, srcp, dstp)

    return pl.pallas_call(
        _tc_final,
        out_shape=jax.ShapeDtypeStruct((n, d_out), jnp.float32),
    )(p2, g2, dinv, b2.reshape(1, d_out))


# speedup vs baseline: 43.1141x; 1.0016x over previous
import functools

import jax
import jax.numpy as jnp
from jax import lax
from jax.experimental import pallas as pl
from jax.experimental.pallas import tpu as pltpu
from jax.experimental.pallas import tpu_sc as plsc

_NC = 2
_NS = 16
_NW = _NC * _NS
_CHUNK = 128
_DEGW = 16


def _round_up(v, m):
    return (v + m - 1) // m * m


def _make_edge_scatter(n, n_pad, ch, d, col_split):
    rpt = n_pad // _NS
    tpt = n // _NS
    mesh = plsc.VectorSubcoreMesh(core_axis_name="c", subcore_axis_name="s")

    @functools.partial(
        pl.kernel,
        out_type=jax.ShapeDtypeStruct((_NC, n_pad, d), jnp.float32),
        mesh=mesh,
        compiler_params=pltpu.CompilerParams(use_tc_tiling_on_sc=False),
        scratch_types=[
            pltpu.VMEM((ch, _CHUNK), jnp.int32),
            pltpu.VMEM((ch, _CHUNK), jnp.int32),
            pltpu.VMEM((4, _CHUNK, d), jnp.float32),
            pltpu.VMEM_SHARED((n_pad, d), jnp.float32),
            pltpu.VMEM_SHARED((n, d), jnp.float32),
            pltpu.SemaphoreType.DMA((4,)),
            pltpu.SemaphoreType.DMA((4,)),
        ],
    )
    def k(table, srcg, dstg, out, src_idx, dst_idx, rows, acc, tbl,
          gsem, ssem):
        cid = lax.axis_index("c")
        sid = lax.axis_index("s")
        gid = sid if col_split else cid * _NS + sid
        pltpu.sync_copy(srcg.at[gid], src_idx)
        pltpu.sync_copy(dstg.at[gid], dst_idx)
        tsrc = table.at[cid] if col_split else table
        pltpu.sync_copy(tsrc.at[pl.ds(sid * tpt, tpt)],
                        tbl.at[pl.ds(sid * tpt, tpt)])
        zv = jnp.zeros((16,), jnp.float32)

        def zstore(i, carry):
            rows[0, i // (d // 16), pl.ds((i % (d // 16)) * 16, 16)] = zv
            return carry

        lax.fori_loop(0, _CHUNK * d // 16, zstore, 0)

        def zcopy(i, carry):
            pltpu.sync_copy(rows.at[0],
                            acc.at[pl.ds(sid * rpt + i * _CHUNK, _CHUNK)])
            return carry

        lax.fori_loop(0, rpt // _CHUNK, zcopy, 0)
        plsc.subcore_barrier()

        pltpu.async_copy(tbl.at[src_idx.at[0]], rows.at[0], gsem.at[0])
        pltpu.async_copy(tbl.at[src_idx.at[1]], rows.at[1], gsem.at[1])

        def step(c, carry):
            bg = lax.rem(c + 2, 4)

            @pl.when(c + 2 < ch)
            def _():
                @pl.when(c >= 2)
                def _():
                    pltpu.make_async_copy(rows.at[bg],
                                          acc.at[dst_idx.at[c]],
                                          ssem.at[bg]).wait()
                pltpu.async_copy(tbl.at[src_idx.at[c + 2]], rows.at[bg],
                                 gsem.at[bg])

            b = lax.rem(c, 4)
            pltpu.make_async_copy(tbl.at[src_idx.at[c]], rows.at[b],
                                  gsem.at[b]).wait()
            pltpu.async_copy(rows.at[b], acc.at[dst_idx.at[c]], ssem.at[b],
                             add=True)
            return carry

        lax.fori_loop(0, ch, step, 0)

        def drain(j, carry):
            pltpu.make_async_copy(rows.at[lax.rem(j, 4)],
                                  acc.at[dst_idx.at[j]],
                                  ssem.at[lax.rem(j, 4)]).wait()
            return carry

        lax.fori_loop(ch - 4, ch, drain, 0)
        plsc.subcore_barrier()
        pltpu.sync_copy(acc.at[pl.ds(sid * rpt, rpt)],
                        out.at[cid, pl.ds(sid * rpt, rpt)])

    return k


def _make_deg_scatter(n_pad, ch):
    rpt = n_pad // _NS
    mesh = plsc.VectorSubcoreMesh(core_axis_name="c", subcore_axis_name="s")

    @functools.partial(
        pl.kernel,
        out_type=jax.ShapeDtypeStruct((_NC, n_pad, _DEGW), jnp.float32),
        mesh=mesh,
        compiler_params=pltpu.CompilerParams(use_tc_tiling_on_sc=False),
        scratch_types=[
            pltpu.VMEM((ch, _CHUNK), jnp.int32),
            pltpu.VMEM((_CHUNK, _DEGW), jnp.float32),
            pltpu.VMEM_SHARED((n_pad, _DEGW), jnp.float32),
            pltpu.SemaphoreType.DMA,
        ],
    )
    def k(dstg, ones, zeros, out, dst_idx, ones_buf, acc, ssem):
        cid = lax.axis_index("c")
        sid = lax.axis_index("s")
        wid = cid * _NS + sid
        pltpu.sync_copy(dstg.at[wid], dst_idx)
        pltpu.sync_copy(ones, ones_buf)
        pltpu.sync_copy(zeros.at[pl.ds(sid * rpt, rpt)],
                        acc.at[pl.ds(sid * rpt, rpt)])
        plsc.subcore_barrier()

        def step(c, carry):
            pltpu.async_copy(ones_buf, acc.at[dst_idx.at[c]], ssem, add=True)
            return carry

        lax.fori_loop(0, ch, step, 0)

        def drain(c, carry):
            pltpu.make_async_copy(ones_buf, acc.at[dst_idx.at[c]],
                                  ssem).wait()
            return carry

        lax.fori_loop(0, ch, drain, 0)
        plsc.subcore_barrier()
        pltpu.sync_copy(acc.at[pl.ds(sid * rpt, rpt)],
                        out.at[cid, pl.ds(sid * rpt, rpt)])

    return k


def _tc_xw(x_ref, w1_ref, xw_ref):
    xw_ref[...] = jnp.dot(x_ref[...], w1_ref[...],
                          preferred_element_type=jnp.float32)


def _tc_scale(degp_ref, xw_ref, g1s_ref, dinv_ref):
    dh = g1s_ref.shape[2]
    deg = degp_ref[0, :, :1] + degp_ref[1, :, :1] + 1.0
    dinv = lax.rsqrt(deg)
    dinv_ref[...] = dinv
    g1 = xw_ref[...] * dinv
    g1s_ref[0] = g1[:, :dh]
    g1s_ref[1] = g1[:, dh:]


def _tc_mid(p_ref, g1s_ref, dinv_ref, b1_ref, w2_ref, g2_ref):
    n = g1s_ref.shape[1]
    s = jnp.concatenate([p_ref[0, :n] + g1s_ref[0],
                         p_ref[1, :n] + g1s_ref[1]], axis=1)
    h = jnp.maximum(s * dinv_ref[...] + b1_ref[...], 0.0)
    g2_ref[...] = jnp.dot(h, w2_ref[...],
                          preferred_element_type=jnp.float32) * dinv_ref[...]


def _tc_final(p_ref, g2_ref, dinv_ref, b2_ref, o_ref):
    n = g2_ref.shape[0]
    o_ref[...] = ((p_ref[0, :n] + p_ref[1, :n] + g2_ref[...])
                  * dinv_ref[...] + b2_ref[...])


def kernel(x, edge_index, W1, b1, W2, b2):
    n, d_in = x.shape
    e = edge_index.shape[1]
    d_h = W1.shape[1]
    d_out = W2.shape[1]
    n_pad = _round_up(n, 256)
    e_pad = _round_up(e, _NW * _CHUNK)
    ch = e_pad // (_NW * _CHUNK)

    src = edge_index[0]
    dst = edge_index[1]
    pad = e_pad - e
    srcp = jnp.concatenate(
        [src, jnp.zeros((pad,), jnp.int32)]).reshape(_NW, ch, _CHUNK)
    dstp = jnp.concatenate(
        [dst, jnp.full((pad,), n, jnp.int32)]).reshape(_NW, ch, _CHUNK)

    ones = jnp.ones((_CHUNK, _DEGW), jnp.float32)
    z_deg = jnp.zeros((n_pad, _DEGW), jnp.float32)

    degp = _make_deg_scatter(n_pad, ch)(dstp, ones, z_deg)

    nb = 2000
    gr = n // nb
    xw = pl.pallas_call(
        _tc_xw,
        grid=(gr,),
        in_specs=[pl.BlockSpec((nb, d_in), lambda i: (i, 0)),
                  pl.BlockSpec((d_in, d_h), lambda i: (0, 0))],
        out_specs=pl.BlockSpec((nb, d_h), lambda i: (i, 0)),
        out_shape=jax.ShapeDtypeStruct((n, d_h), jnp.float32),
    )(x, W1)

    g1s, dinv = pl.pallas_call(
        _tc_scale,
        grid=(gr,),
        in_specs=[pl.BlockSpec((2, nb, _DEGW), lambda i: (0, i, 0)),
                  pl.BlockSpec((nb, d_h), lambda i: (i, 0))],
        out_specs=(pl.BlockSpec((2, nb, d_h // 2), lambda i: (0, i, 0)),
                   pl.BlockSpec((nb, 1), lambda i: (i, 0))),
        out_shape=(jax.ShapeDtypeStruct((2, n, d_h // 2), jnp.float32),
                   jax.ShapeDtypeStruct((n, 1), jnp.float32)),
    )(degp, xw)

    srcc = srcp.reshape(_NS, 2 * ch, _CHUNK)
    dstc = dstp.reshape(_NS, 2 * ch, _CHUNK)
    p1 = _make_edge_scatter(n, n_pad, 2 * ch, d_h // 2, True)(g1s, srcc, dstc)

    g2 = pl.pallas_call(
        _tc_mid,
        grid=(gr,),
        in_specs=[pl.BlockSpec((2, nb, d_h // 2), lambda i: (0, i, 0)),
                  pl.BlockSpec((2, nb, d_h // 2), lambda i: (0, i, 0)),
                  pl.BlockSpec((nb, 1), lambda i: (i, 0)),
                  pl.BlockSpec((1, d_h), lambda i: (0, 0)),
                  pl.BlockSpec((d_h, d_out), lambda i: (0, 0))],
        out_specs=pl.BlockSpec((nb, d_out), lambda i: (i, 0)),
        out_shape=jax.ShapeDtypeStruct((n, d_out), jnp.float32),
    )(p1, g1s, dinv, b1.reshape(1, d_h), W2)

    p2 = _make_edge_scatter(n, n_pad, ch, d_out, False)(g2, srcp, dstp)

    return pl.pallas_call(
        _tc_final,
        grid=(gr,),
        in_specs=[pl.BlockSpec((2, nb, d_out), lambda i: (0, i, 0)),
                  pl.BlockSpec((nb, d_out), lambda i: (i, 0)),
                  pl.BlockSpec((nb, 1), lambda i: (i, 0)),
                  pl.BlockSpec((1, d_out), lambda i: (0, 0))],
        out_specs=pl.BlockSpec((nb, d_out), lambda i: (i, 0)),
        out_shape=jax.ShapeDtypeStruct((n, d_out), jnp.float32),
    )(p2, g2, dinv, b2.reshape(1, d_out))


# speedup vs baseline: 43.5813x; 1.0108x over previous
import functools

import jax
import jax.numpy as jnp
from jax import lax
from jax.experimental import pallas as pl
from jax.experimental.pallas import tpu as pltpu
from jax.experimental.pallas import tpu_sc as plsc

_NC = 2
_NS = 16
_NW = _NC * _NS
_CHUNK = 128
_DEGW = 8
_RING = 12
_LEAD = 4


def _round_up(v, m):
    return (v + m - 1) // m * m


def _make_edge_scatter(n, n_pad, ch, d, col_split):
    rpt = n_pad // _NS
    tpt = n // _NS
    mesh = plsc.VectorSubcoreMesh(core_axis_name="c", subcore_axis_name="s")

    @functools.partial(
        pl.kernel,
        out_type=jax.ShapeDtypeStruct((_NC, n_pad, d), jnp.float32),
        mesh=mesh,
        compiler_params=pltpu.CompilerParams(use_tc_tiling_on_sc=False),
        scratch_types=[
            pltpu.VMEM((ch, _CHUNK), jnp.int32),
            pltpu.VMEM((ch, _CHUNK), jnp.int32),
            pltpu.VMEM((_RING, _CHUNK, d), jnp.float32),
            pltpu.VMEM_SHARED((n_pad, d), jnp.float32),
            pltpu.VMEM_SHARED((n, d), jnp.float32),
            pltpu.SemaphoreType.DMA((_RING,)),
            pltpu.SemaphoreType.DMA((_RING,)),
        ],
    )
    def k(table, srcg, dstg, out, src_idx, dst_idx, rows, acc, tbl,
          gsem, ssem):
        cid = lax.axis_index("c")
        sid = lax.axis_index("s")
        gid = sid if col_split else cid * _NS + sid
        pltpu.sync_copy(srcg.at[gid], src_idx)
        pltpu.sync_copy(dstg.at[gid], dst_idx)
        tsrc = table.at[cid] if col_split else table
        pltpu.sync_copy(tsrc.at[pl.ds(sid * tpt, tpt)],
                        tbl.at[pl.ds(sid * tpt, tpt)])
        zv = jnp.zeros((16,), jnp.float32)

        def zstore(i, carry):
            rows[0, i // (d // 16), pl.ds((i % (d // 16)) * 16, 16)] = zv
            return carry

        lax.fori_loop(0, _CHUNK * d // 16, zstore, 0)

        def zcopy(i, carry):
            pltpu.sync_copy(rows.at[0],
                            acc.at[pl.ds(sid * rpt + i * _CHUNK, _CHUNK)])
            return carry

        lax.fori_loop(0, rpt // _CHUNK, zcopy, 0)
        plsc.subcore_barrier()

        for b0 in range(_LEAD):
            pltpu.async_copy(tbl.at[src_idx.at[b0]], rows.at[b0],
                             gsem.at[b0])

        def step(c, carry):
            bg = lax.rem(c + _LEAD, _RING)

            @pl.when(c + _LEAD < ch)
            def _():
                @pl.when(c >= _RING - _LEAD)
                def _():
                    pltpu.make_async_copy(rows.at[bg],
                                          acc.at[dst_idx.at[c]],
                                          ssem.at[bg]).wait()
                pltpu.async_copy(tbl.at[src_idx.at[c + _LEAD]], rows.at[bg],
                                 gsem.at[bg])

            b = lax.rem(c, _RING)
            pltpu.make_async_copy(tbl.at[src_idx.at[c]], rows.at[b],
                                  gsem.at[b]).wait()
            pltpu.async_copy(rows.at[b], acc.at[dst_idx.at[c]], ssem.at[b],
                             add=True)
            return carry

        lax.fori_loop(0, ch, step, 0)

        def drain(j, carry):
            pltpu.make_async_copy(rows.at[lax.rem(j, _RING)],
                                  acc.at[dst_idx.at[j]],
                                  ssem.at[lax.rem(j, _RING)]).wait()
            return carry

        lax.fori_loop(ch - _RING, ch, drain, 0)
        plsc.subcore_barrier()
        pltpu.sync_copy(acc.at[pl.ds(sid * rpt, rpt)],
                        out.at[cid, pl.ds(sid * rpt, rpt)])

    return k


def _make_deg_scatter(n_pad, ch):
    rpt = n_pad // _NS
    mesh = plsc.VectorSubcoreMesh(core_axis_name="c", subcore_axis_name="s")

    @functools.partial(
        pl.kernel,
        out_type=jax.ShapeDtypeStruct((_NC, n_pad, _DEGW), jnp.float32),
        mesh=mesh,
        compiler_params=pltpu.CompilerParams(use_tc_tiling_on_sc=False),
        scratch_types=[
            pltpu.VMEM((ch, _CHUNK), jnp.int32),
            pltpu.VMEM((_CHUNK, _DEGW), jnp.float32),
            pltpu.VMEM_SHARED((n_pad, _DEGW), jnp.float32),
            pltpu.SemaphoreType.DMA,
        ],
    )
    def k(dstg, ones, zeros, out, dst_idx, ones_buf, acc, ssem):
        cid = lax.axis_index("c")
        sid = lax.axis_index("s")
        wid = cid * _NS + sid
        pltpu.sync_copy(dstg.at[wid], dst_idx)
        pltpu.sync_copy(ones, ones_buf)
        pltpu.sync_copy(zeros.at[pl.ds(sid * rpt, rpt)],
                        acc.at[pl.ds(sid * rpt, rpt)])
        plsc.subcore_barrier()

        def step(c, carry):
            pltpu.async_copy(ones_buf, acc.at[dst_idx.at[c]], ssem, add=True)
            return carry

        lax.fori_loop(0, ch, step, 0)

        def drain(c, carry):
            pltpu.make_async_copy(ones_buf, acc.at[dst_idx.at[c]],
                                  ssem).wait()
            return carry

        lax.fori_loop(0, ch, drain, 0)
        plsc.subcore_barrier()
        pltpu.sync_copy(acc.at[pl.ds(sid * rpt, rpt)],
                        out.at[cid, pl.ds(sid * rpt, rpt)])

    return k


def _tc_xw(x_ref, w1_ref, xw_ref):
    xw_ref[...] = jnp.dot(x_ref[...], w1_ref[...],
                          preferred_element_type=jnp.float32)


def _tc_scale(degp_ref, xw_ref, g1s_ref, dinv_ref):
    dh = g1s_ref.shape[2]
    deg = degp_ref[0, :, :1] + degp_ref[1, :, :1] + 1.0
    dinv = lax.rsqrt(deg)
    dinv_ref[...] = dinv
    g1 = xw_ref[...] * dinv
    g1s_ref[0] = g1[:, :dh]
    g1s_ref[1] = g1[:, dh:]


def _tc_mid(p_ref, g1s_ref, dinv_ref, b1_ref, w2_ref, g2_ref):
    n = g1s_ref.shape[1]
    s = jnp.concatenate([p_ref[0, :n] + g1s_ref[0],
                         p_ref[1, :n] + g1s_ref[1]], axis=1)
    h = jnp.maximum(s * dinv_ref[...] + b1_ref[...], 0.0)
    g2_ref[...] = jnp.dot(h, w2_ref[...],
                          preferred_element_type=jnp.float32) * dinv_ref[...]


def _tc_final(p_ref, g2_ref, dinv_ref, b2_ref, o_ref):
    n = g2_ref.shape[0]
    o_ref[...] = ((p_ref[0, :n] + p_ref[1, :n] + g2_ref[...])
                  * dinv_ref[...] + b2_ref[...])


def kernel(x, edge_index, W1, b1, W2, b2):
    n, d_in = x.shape
    e = edge_index.shape[1]
    d_h = W1.shape[1]
    d_out = W2.shape[1]
    n_pad = _round_up(n, 256)
    e_pad = _round_up(e, _NW * _CHUNK)
    ch = e_pad // (_NW * _CHUNK)

    src = edge_index[0]
    dst = edge_index[1]
    pad = e_pad - e
    srcp = jnp.concatenate(
        [src, jnp.zeros((pad,), jnp.int32)]).reshape(_NW, ch, _CHUNK)
    dstp = jnp.concatenate(
        [dst, jnp.full((pad,), n, jnp.int32)]).reshape(_NW, ch, _CHUNK)

    ones = jnp.ones((_CHUNK, _DEGW), jnp.float32)
    z_deg = jnp.zeros((n_pad, _DEGW), jnp.float32)

    degp = _make_deg_scatter(n_pad, ch)(dstp, ones, z_deg)

    nb = 2000
    gr = n // nb
    xw = pl.pallas_call(
        _tc_xw,
        grid=(gr,),
        in_specs=[pl.BlockSpec((nb, d_in), lambda i: (i, 0)),
                  pl.BlockSpec((d_in, d_h), lambda i: (0, 0))],
        out_specs=pl.BlockSpec((nb, d_h), lambda i: (i, 0)),
        out_shape=jax.ShapeDtypeStruct((n, d_h), jnp.float32),
    )(x, W1)

    g1s, dinv = pl.pallas_call(
        _tc_scale,
        grid=(gr,),
        in_specs=[pl.BlockSpec((2, nb, _DEGW), lambda i: (0, i, 0)),
                  pl.BlockSpec((nb, d_h), lambda i: (i, 0))],
        out_specs=(pl.BlockSpec((2, nb, d_h // 2), lambda i: (0, i, 0)),
                   pl.BlockSpec((nb, 1), lambda i: (i, 0))),
        out_shape=(jax.ShapeDtypeStruct((2, n, d_h // 2), jnp.float32),
                   jax.ShapeDtypeStruct((n, 1), jnp.float32)),
    )(degp, xw)

    srcc = srcp.reshape(_NS, 2 * ch, _CHUNK)
    dstc = dstp.reshape(_NS, 2 * ch, _CHUNK)
    p1 = _make_edge_scatter(n, n_pad, 2 * ch, d_h // 2, True)(g1s, srcc, dstc)

    g2 = pl.pallas_call(
        _tc_mid,
        grid=(gr,),
        in_specs=[pl.BlockSpec((2, nb, d_h // 2), lambda i: (0, i, 0)),
                  pl.BlockSpec((2, nb, d_h // 2), lambda i: (0, i, 0)),
                  pl.BlockSpec((nb, 1), lambda i: (i, 0)),
                  pl.BlockSpec((1, d_h), lambda i: (0, 0)),
                  pl.BlockSpec((d_h, d_out), lambda i: (0, 0))],
        out_specs=pl.BlockSpec((nb, d_out), lambda i: (i, 0)),
        out_shape=jax.ShapeDtypeStruct((n, d_out), jnp.float32),
    )(p1, g1s, dinv, b1.reshape(1, d_h), W2)

    p2 = _make_edge_scatter(n, n_pad, ch, d_out, False)(g2, srcp, dstp)

    return pl.pallas_call(
        _tc_final,
        grid=(gr,),
        in_specs=[pl.BlockSpec((2, nb, d_out), lambda i: (0, i, 0)),
                  pl.BlockSpec((nb, d_out), lambda i: (i, 0)),
                  pl.BlockSpec((nb, 1), lambda i: (i, 0)),
                  pl.BlockSpec((1, d_out), lambda i: (0, 0))],
        out_specs=pl.BlockSpec((nb, d_out), lambda i: (i, 0)),
        out_shape=jax.ShapeDtypeStruct((n, d_out), jnp.float32),
    )(p2, g2, dinv, b2.reshape(1, d_out))
